# baseline (speedup 1.0000x reference)
import math

import jax
import jax.numpy as jnp
from jax.experimental import pallas as pl
from jax.experimental.pallas import tpu as pltpu

D = 64
B = 16384
_HALF = D // 2

_TA = (((0,), (0,)), ((), ()))


def _fused_kernel(xT_ref, cT_ref, t_ref, Wstep_ref, W1_ref, W2T_ref,
                  bstep_ref, b1_ref, b2_ref, outT_ref,
                  W1aT_ref, W1bT_ref, WcT_ref, b1c_ref, b2c_ref):
    f32 = jnp.float32

    @pl.when(pl.program_id(0) == 0)
    def _prep():
        W1c = W1_ref[2 * D:3 * D, :]
        W1aT_ref[:] = W1_ref[0:D, :].T
        W1bT_ref[:] = W1_ref[D:2 * D, :].T
        Wc = jnp.dot(Wstep_ref[:], W1c, preferred_element_type=f32)
        WcT_ref[:] = Wc.T
        bc = jax.lax.dot_general(W1c, bstep_ref[:], (((0,), (1,)), ((), ())),
                                 preferred_element_type=f32)
        ones1 = jnp.ones((1, 1), f32)
        b1c_ref[:] = bc + jax.lax.dot_general(b1_ref[:], ones1, _TA,
                                              preferred_element_type=f32)
        b2c_ref[:] = jax.lax.dot_general(b2_ref[:], ones1, _TA,
                                         preferred_element_type=f32)

    t_row = t_ref[:].astype(f32)
    row = jax.lax.broadcasted_iota(jnp.int32, (D, 1), 0)
    fidx = jnp.where(row < _HALF, row, row - _HALF).astype(f32)
    freq_col = jnp.exp(fidx * (-math.log(10000.0) / _HALF))
    shift_col = jnp.where(row < _HALF, 0.0, math.pi / 2).astype(f32)
    args = freq_col * t_row - shift_col
    n = jnp.round(args * f32(1.0 / (2.0 * math.pi)))
    r = args - n * f32(6.28125)
    r = r - n * f32(1.9353071795864769e-03)
    s = r * r
    tT = f32(-2.2398469402767916e-07)
    tT = tT * s + f32(2.430807671139143e-05)
    tT = tT * s + f32(-1.3867885560937686e-03)
    tT = tT * s + f32(4.1662991555676473e-02)
    tT = tT * s + f32(-4.999981914909368e-01)
    tT = tT * s + f32(1.0)

    h = jnp.dot(W1aT_ref[:], xT_ref[:], preferred_element_type=f32)
    h += jnp.dot(W1bT_ref[:], cT_ref[:], preferred_element_type=f32)
    h += jnp.dot(WcT_ref[:], tT, preferred_element_type=f32)
    h += b1c_ref[:]
    h = jnp.where(h > 0, h, 0.01 * h)

    o = jnp.dot(W2T_ref[:], h, preferred_element_type=f32)
    outT_ref[:] = o + b2c_ref[:]


@jax.jit
def kernel(x, t, c, W_step, b_step, W1, b1, W2, b2):
    bx = 1024
    grid = (B // bx,)

    xT = x.T
    cT = c.T
    W2T = W2.T
    t2 = t.reshape(1, B)

    outT = pl.pallas_call(
        _fused_kernel,
        grid=grid,
        in_specs=[
            pl.BlockSpec((D, bx), lambda i: (0, i)),
            pl.BlockSpec((D, bx), lambda i: (0, i)),
            pl.BlockSpec((1, bx), lambda i: (0, i)),
            pl.BlockSpec((D, D), lambda i: (0, 0)),
            pl.BlockSpec((3 * D, 3 * D), lambda i: (0, 0)),
            pl.BlockSpec((D, 3 * D), lambda i: (0, 0)),
            pl.BlockSpec((1, D), lambda i: (0, 0)),
            pl.BlockSpec((1, 3 * D), lambda i: (0, 0)),
            pl.BlockSpec((1, D), lambda i: (0, 0)),
        ],
        out_specs=pl.BlockSpec((D, bx), lambda i: (0, i)),
        out_shape=jax.ShapeDtypeStruct((D, B), jnp.float32),
        compiler_params=pltpu.CompilerParams(
            dimension_semantics=("arbitrary",),
        ),
        scratch_shapes=[
            pltpu.VMEM((3 * D, D), jnp.float32),
            pltpu.VMEM((3 * D, D), jnp.float32),
            pltpu.VMEM((3 * D, D), jnp.float32),
            pltpu.VMEM((3 * D, 1), jnp.float32),
            pltpu.VMEM((D, 1), jnp.float32),
        ],
    )(xT, cT, t2, W_step, W1, W2T,
      b_step.reshape(1, D), b1.reshape(1, 3 * D), b2.reshape(1, D))
    return outT.T


# speedup vs baseline: 1.3264x; 1.3264x over previous
import math

import jax
import jax.numpy as jnp
from jax.experimental import pallas as pl
from jax.experimental.pallas import tpu as pltpu

D = 64
B = 16384
_HALF = D // 2
_KA = D + 8

_TA = (((0,), (0,)), ((), ()))


def _fused_kernel(xT_ref, cT_ref, t_ref, Wstep_ref, W1_ref, W2T_ref,
                  bstep_ref, b1_ref, b2_ref, outT_ref,
                  W1aT_ref, W1bT_ref, WcTa_ref, tTa_ref, b2c_ref):
    f32 = jnp.float32
    bx = t_ref.shape[1]

    @pl.when(pl.program_id(0) == 0)
    def _prep():
        W1c = W1_ref[2 * D:3 * D, :]
        W1aT_ref[:] = W1_ref[0:D, :].T
        W1bT_ref[:] = W1_ref[D:2 * D, :].T
        Wc = jnp.dot(Wstep_ref[:], W1c, preferred_element_type=f32)
        bc = jax.lax.dot_general(W1c, bstep_ref[:], (((0,), (1,)), ((), ())),
                                 preferred_element_type=f32)
        ones1 = jnp.ones((1, 1), f32)
        b1c = bc + jax.lax.dot_general(b1_ref[:], ones1, _TA,
                                       preferred_element_type=f32)
        pad = jax.lax.broadcasted_iota(jnp.int32, (3 * D, 8), 1)
        WcTa_ref[:] = jnp.concatenate(
            [Wc.T, jnp.where(pad == 0, b1c, 0.0)], axis=1)
        aug = jax.lax.broadcasted_iota(jnp.int32, (8, bx), 0)
        tTa_ref[D:_KA, :] = jnp.where(aug == 0, 1.0, 0.0)
        b2c_ref[:] = jax.lax.dot_general(b2_ref[:], ones1, _TA,
                                         preferred_element_type=f32)

    t_row = t_ref[:].astype(f32)
    row = jax.lax.broadcasted_iota(jnp.int32, (D, 1), 0)
    fidx = jnp.where(row < _HALF, row, row - _HALF).astype(f32)
    freq_col = jnp.exp(fidx * (-math.log(10000.0) / _HALF))
    shift_col = jnp.where(row < _HALF, 0.0, math.pi / 2).astype(f32)
    args = freq_col * t_row - shift_col
    n = jnp.round(args * f32(1.0 / (2.0 * math.pi)))
    r = args - n * f32(6.28125)
    r = r - n * f32(1.9353071795864769e-03)
    s = r * r
    tT = f32(1.9323491007983232e-05)
    tT = tT * s + f32(-1.3497862134739707e-03)
    tT = tT * s + f32(4.1561140783182654e-02)
    tT = tT * s + f32(-4.999223439204043e-01)
    tT = tT * s + f32(1.0)
    tTa_ref[0:D, :] = tT

    h = jnp.dot(W1aT_ref[:], xT_ref[:], preferred_element_type=f32)
    h += jnp.dot(W1bT_ref[:], cT_ref[:], preferred_element_type=f32)
    h += jnp.dot(WcTa_ref[:], tTa_ref[:], preferred_element_type=f32)
    h = jnp.where(h > 0, h, 0.01 * h)

    o = jnp.dot(W2T_ref[:], h, preferred_element_type=f32)
    outT_ref[:] = o + b2c_ref[:]


@jax.jit
def kernel(x, t, c, W_step, b_step, W1, b1, W2, b2):
    bx = 2048
    grid = (B // bx,)

    xT = x.T
    cT = c.T
    W2T = W2.T
    t2 = t.reshape(1, B)

    outT = pl.pallas_call(
        _fused_kernel,
        grid=grid,
        in_specs=[
            pl.BlockSpec((D, bx), lambda i: (0, i)),
            pl.BlockSpec((D, bx), lambda i: (0, i)),
            pl.BlockSpec((1, bx), lambda i: (0, i)),
            pl.BlockSpec((D, D), lambda i: (0, 0)),
            pl.BlockSpec((3 * D, 3 * D), lambda i: (0, 0)),
            pl.BlockSpec((D, 3 * D), lambda i: (0, 0)),
            pl.BlockSpec((1, D), lambda i: (0, 0)),
            pl.BlockSpec((1, 3 * D), lambda i: (0, 0)),
            pl.BlockSpec((1, D), lambda i: (0, 0)),
        ],
        out_specs=pl.BlockSpec((D, bx), lambda i: (0, i)),
        out_shape=jax.ShapeDtypeStruct((D, B), jnp.float32),
        scratch_shapes=[
            pltpu.VMEM((3 * D, D), jnp.float32),
            pltpu.VMEM((3 * D, D), jnp.float32),
            pltpu.VMEM((3 * D, _KA), jnp.float32),
            pltpu.VMEM((_KA, bx), jnp.float32),
            pltpu.VMEM((D, 1), jnp.float32),
        ],
    )(xT, cT, t2, W_step, W1, W2T,
      b_step.reshape(1, D), b1.reshape(1, 3 * D), b2.reshape(1, D))
    return outT.T


# speedup vs baseline: 1.3837x; 1.0432x over previous
import math

import jax
import jax.numpy as jnp
from jax.experimental import pallas as pl
from jax.experimental.pallas import tpu as pltpu

D = 64
B = 16384
_HALF = D // 2
_KA = D + 8

_TA = (((0,), (0,)), ((), ()))


def _fused_kernel(xT_ref, cT_ref, t_ref, Wstep_ref, W1_ref, W2T_ref,
                  bstep_ref, b1_ref, b2_ref, outT_ref,
                  W1aT_ref, W1bT_ref, WcTa_ref, tTa_ref, b2c_ref):
    f32 = jnp.float32
    bx = t_ref.shape[1]

    @pl.when(pl.program_id(0) == 0)
    def _prep():
        W1c = W1_ref[2 * D:3 * D, :]
        W1aT_ref[:] = W1_ref[0:D, :].T
        W1bT_ref[:] = W1_ref[D:2 * D, :].T
        Wc = jnp.dot(Wstep_ref[:], W1c, preferred_element_type=f32)
        bc = jax.lax.dot_general(W1c, bstep_ref[:], (((0,), (1,)), ((), ())),
                                 preferred_element_type=f32)
        ones1 = jnp.ones((1, 1), f32)
        b1c = bc + jax.lax.dot_general(b1_ref[:], ones1, _TA,
                                       preferred_element_type=f32)
        pad = jax.lax.broadcasted_iota(jnp.int32, (3 * D, 8), 1)
        WcTa_ref[:] = jnp.concatenate(
            [Wc.T, jnp.where(pad == 0, b1c, 0.0)], axis=1)
        aug = jax.lax.broadcasted_iota(jnp.int32, (8, bx), 0)
        tTa_ref[D:_KA, :] = jnp.where(aug == 0, 1.0, 0.0)
        b2c_ref[:] = jax.lax.dot_general(b2_ref[:], ones1, _TA,
                                         preferred_element_type=f32)

    t_row = t_ref[:].astype(f32)
    row = jax.lax.broadcasted_iota(jnp.int32, (D, 1), 0)
    fidx = jnp.where(row < _HALF, row, row - _HALF).astype(f32)
    fs_col = jnp.exp(fidx * (-math.log(10000.0) / _HALF)) \
        * f32(1.0 / (2.0 * math.pi))
    sh_col = jnp.where(row < _HALF, 0.0, 0.25).astype(f32)
    u = fs_col * t_row - sh_col
    v = u - jnp.round(u)
    s = v * v
    tT = f32(46.93799520461047)
    tT = tT * s + f32(-83.05087227333142)
    tT = tT * s + f32(64.77492713641647)
    tT = tT * s + f32(-19.736143063038917)
    tT = tT * s + f32(1.0)
    tTa_ref[0:D, :] = tT

    h = jnp.dot(W1aT_ref[:], xT_ref[:], preferred_element_type=f32)
    h += jnp.dot(W1bT_ref[:], cT_ref[:], preferred_element_type=f32)
    h += jnp.dot(WcTa_ref[:], tTa_ref[:], preferred_element_type=f32)
    h = jnp.where(h > 0, h, 0.01 * h)

    o = jnp.dot(W2T_ref[:], h, preferred_element_type=f32)
    outT_ref[:] = o + b2c_ref[:]


@jax.jit
def kernel(x, t, c, W_step, b_step, W1, b1, W2, b2):
    bx = 2048
    grid = (B // bx,)

    xT = x.T
    cT = c.T
    W2T = W2.T
    t2 = t.reshape(1, B)

    outT = pl.pallas_call(
        _fused_kernel,
        grid=grid,
        in_specs=[
            pl.BlockSpec((D, bx), lambda i: (0, i)),
            pl.BlockSpec((D, bx), lambda i: (0, i)),
            pl.BlockSpec((1, bx), lambda i: (0, i)),
            pl.BlockSpec((D, D), lambda i: (0, 0)),
            pl.BlockSpec((3 * D, 3 * D), lambda i: (0, 0)),
            pl.BlockSpec((D, 3 * D), lambda i: (0, 0)),
            pl.BlockSpec((1, D), lambda i: (0, 0)),
            pl.BlockSpec((1, 3 * D), lambda i: (0, 0)),
            pl.BlockSpec((1, D), lambda i: (0, 0)),
        ],
        out_specs=pl.BlockSpec((D, bx), lambda i: (0, i)),
        out_shape=jax.ShapeDtypeStruct((D, B), jnp.float32),
        scratch_shapes=[
            pltpu.VMEM((3 * D, D), jnp.float32),
            pltpu.VMEM((3 * D, D), jnp.float32),
            pltpu.VMEM((3 * D, _KA), jnp.float32),
            pltpu.VMEM((_KA, bx), jnp.float32),
            pltpu.VMEM((D, 1), jnp.float32),
        ],
    )(xT, cT, t2, W_step, W1, W2T,
      b_step.reshape(1, D), b1.reshape(1, 3 * D), b2.reshape(1, D))
    return outT.T


# speedup vs baseline: 1.3848x; 1.0008x over previous
import math

import jax
import jax.numpy as jnp
from jax.experimental import pallas as pl
from jax.experimental.pallas import tpu as pltpu

D = 64
B = 16384
_HALF = D // 2
_KA = D + 8

_TA = (((0,), (0,)), ((), ()))


def _fused_kernel(xT_ref, cT_ref, t_ref, Wstep_ref, W1_ref, W2T_ref,
                  bstep_ref, b1_ref, b2_ref, outT_ref,
                  W1aT_ref, W1bT_ref, WcTa_ref, tTa_ref, b2c_ref):
    f32 = jnp.float32
    bx = t_ref.shape[1]

    @pl.when(pl.program_id(0) == 0)
    def _prep():
        W1c = W1_ref[2 * D:3 * D, :]
        W1aT_ref[:] = W1_ref[0:D, :].T
        W1bT_ref[:] = W1_ref[D:2 * D, :].T
        Wc = jnp.dot(Wstep_ref[:], W1c, preferred_element_type=f32)
        bc = jax.lax.dot_general(W1c, bstep_ref[:], (((0,), (1,)), ((), ())),
                                 preferred_element_type=f32)
        ones1 = jnp.ones((1, 1), f32)
        b1c = bc + jax.lax.dot_general(b1_ref[:], ones1, _TA,
                                       preferred_element_type=f32)
        pad = jax.lax.broadcasted_iota(jnp.int32, (3 * D, 8), 1)
        WcTa_ref[:] = jnp.concatenate(
            [Wc.T, jnp.where(pad == 0, b1c, 0.0)], axis=1)
        aug = jax.lax.broadcasted_iota(jnp.int32, (8, bx), 0)
        tTa_ref[D:_KA, :] = jnp.where(aug == 0, 1.0, 0.0)
        b2c_ref[:] = jax.lax.dot_general(b2_ref[:], ones1, _TA,
                                         preferred_element_type=f32)

    t_row = t_ref[:].astype(f32)
    row = jax.lax.broadcasted_iota(jnp.int32, (D, 1), 0)
    fidx = jnp.where(row < _HALF, row, row - _HALF).astype(f32)
    fs_col = jnp.exp(fidx * (-math.log(10000.0) / _HALF)) \
        * f32(1.0 / (2.0 * math.pi))
    sh_col = jnp.where(row < _HALF, 0.0, 0.25).astype(f32)
    u = fs_col * t_row - sh_col
    v = u - jnp.round(u)
    s = v * v
    tT = f32(46.93799520461047)
    tT = tT * s + f32(-83.05087227333142)
    tT = tT * s + f32(64.77492713641647)
    tT = tT * s + f32(-19.736143063038917)
    tT = tT * s + f32(1.0)
    tTa_ref[0:D, :] = tT

    h = jnp.dot(W1aT_ref[:], xT_ref[:], preferred_element_type=f32)
    h += jnp.dot(W1bT_ref[:], cT_ref[:], preferred_element_type=f32)
    h += jnp.dot(WcTa_ref[:], tTa_ref[:], preferred_element_type=f32)
    h = jnp.where(h > 0, h, 0.01 * h)

    o = jnp.dot(W2T_ref[:], h, preferred_element_type=f32)
    outT_ref[:] = o + b2c_ref[:]


@jax.jit
def kernel(x, t, c, W_step, b_step, W1, b1, W2, b2):
    bx = 2048
    grid = (B // bx,)

    xT = x.T
    cT = c.T
    W2T = W2.T
    t2 = t.reshape(1, B)

    outT = pl.pallas_call(
        _fused_kernel,
        grid=grid,
        in_specs=[
            pl.BlockSpec((D, bx), lambda i: (0, i),
                         pipeline_mode=pl.Buffered(2)),
            pl.BlockSpec((D, bx), lambda i: (0, i),
                         pipeline_mode=pl.Buffered(2)),
            pl.BlockSpec((1, bx), lambda i: (0, i),
                         pipeline_mode=pl.Buffered(2)),
            pl.BlockSpec((D, D), lambda i: (0, 0)),
            pl.BlockSpec((3 * D, 3 * D), lambda i: (0, 0)),
            pl.BlockSpec((D, 3 * D), lambda i: (0, 0)),
            pl.BlockSpec((1, D), lambda i: (0, 0)),
            pl.BlockSpec((1, 3 * D), lambda i: (0, 0)),
            pl.BlockSpec((1, D), lambda i: (0, 0)),
        ],
        out_specs=pl.BlockSpec((D, bx), lambda i: (0, i),
                               pipeline_mode=pl.Buffered(2)),
        out_shape=jax.ShapeDtypeStruct((D, B), jnp.float32),
        scratch_shapes=[
            pltpu.VMEM((3 * D, D), jnp.float32),
            pltpu.VMEM((3 * D, D), jnp.float32),
            pltpu.VMEM((3 * D, _KA), jnp.float32),
            pltpu.VMEM((_KA, bx), jnp.float32),
            pltpu.VMEM((D, 1), jnp.float32),
        ],
    )(xT, cT, t2, W_step, W1, W2T,
      b_step.reshape(1, D), b1.reshape(1, 3 * D), b2.reshape(1, D))
    return outT.T
